# baseline (speedup 1.0000x reference)
import functools

import jax
import jax.numpy as jnp
from jax import lax
from jax.experimental import pallas as pl
from jax.experimental.pallas import tpu as pltpu
from jax.experimental.pallas import tpu_sc as plsc

B, F, V, D = 16384, 26, 8, 128
NC, NS = 2, 16
NW = NC * NS
ROWS = B * F
RPW = ROWS // NW
CH = 128
NCH = RPW // CH
NB = 4
NBANDS = B // 8


@functools.partial(
    pl.kernel,
    out_type=jax.ShapeDtypeStruct((NBANDS, F, 8, D), jnp.float32),
    mesh=plsc.VectorSubcoreMesh(core_axis_name="c", subcore_axis_name="s"),
    scratch_types=(
        [pltpu.VMEM((NCH, CH), jnp.int32)]
        + [pltpu.VMEM((CH, D), jnp.float32) for _ in range(NB)]
        + [pltpu.VMEM_SHARED((NS * F * V, D), jnp.float32)]
        + [pltpu.SemaphoreType.DMA for _ in range(2 * NB)]
    ),
)
def _sc_lookup(tbl_hbm, idx_hbm, out_4d, idx_v, *rest):
    out_hbm = out_4d.reshape(ROWS, D)
    bufs = rest[:NB]
    tbl_sh = rest[NB]
    gsem = rest[NB + 1:2 * NB + 1]
    ssem = rest[2 * NB + 1:]

    sid = lax.axis_index("s")
    wid = sid * NC + lax.axis_index("c")

    pltpu.sync_copy(tbl_hbm, tbl_sh.at[pl.ds(sid * F * V, F * V)])
    pltpu.sync_copy(idx_hbm.at[wid], idx_v)

    tbl_base = sid * F * V

    def add_base(j, carry):
        for t in range(CH // 16):
            sl = pl.ds(t * 16, 16)
            idx_v[j, sl] = idx_v[j, sl] + tbl_base
        return carry

    lax.fori_loop(0, NCH, add_base, 0)
    plsc.subcore_barrier()

    base = wid * RPW

    def start_gather(g, p):
        pltpu.async_copy(tbl_sh.at[idx_v.at[g]], bufs[p], gsem[p])

    def wait_gather(g, p):
        pltpu.make_async_copy(tbl_sh.at[idx_v.at[g]], bufs[p], gsem[p]).wait()

    def start_scatter(g, p):
        pltpu.async_copy(bufs[p], out_hbm.at[pl.ds(base + g * CH, CH)], ssem[p])

    def wait_scatter(g, p):
        pltpu.make_async_copy(bufs[p], out_hbm.at[pl.ds(base + g * CH, CH)],
                              ssem[p]).wait()

    for p in range(NB):
        start_gather(p, p)

    def body(k, carry):
        g = NB * k
        for p in range(NB):
            wait_gather(g + p, p)
            start_scatter(g + p, p)
        for p in range(NB):
            wait_scatter(g + p, p)
            start_gather(g + NB + p, p)
        return carry

    lax.fori_loop(0, NCH // NB - 1, body, 0)

    g = NCH - NB
    for p in range(NB):
        wait_gather(g + p, p)
        start_scatter(g + p, p)
    for p in range(NB):
        wait_scatter(g + p, p)


def kernel(indices, tables):
    tbl = tables.reshape(F * V, D)
    flat_idx = indices + jnp.arange(F, dtype=jnp.int32)[None, :] * V
    perm_idx = flat_idx.reshape(NBANDS, 8, F).transpose(0, 2, 1)
    idx3 = perm_idx.reshape(NW, NCH, CH)
    out = _sc_lookup(tbl, idx3)
    return out.transpose(0, 2, 1, 3).reshape(B, F * D)
